# baseline (speedup 1.0000x reference)
import functools

import jax
import jax.numpy as jnp
from jax import lax
from jax.experimental import pallas as pl
from jax.experimental.pallas import tpu as pltpu
from jax.experimental.pallas import tpu_sc as plsc

_B, _S, _D = 16, 2048, 64
_N = _B * _S
_NW = 32
_RPW = _N // _NW
_CHUNK = 512
_NCHUNK = _RPW // _CHUNK
_G = 128
_NG = _CHUNK // _G
_L = 16

_mesh = plsc.VectorSubcoreMesh(core_axis_name="c", subcore_axis_name="s")


@functools.partial(
    pl.kernel,
    mesh=_mesh,
    out_type=jax.ShapeDtypeStruct((_N // 2, 128), jnp.float32),
    scratch_types=[
        pltpu.VMEM((_RPW // _G, _G), jnp.int32),
        pltpu.VMEM((_RPW // _G, _G), jnp.int32),
        pltpu.VMEM((_CHUNK, 128), jnp.float32),
        pltpu.VMEM((_CHUNK // 2, 128), jnp.float32),
        pltpu.SemaphoreType.DMA,
    ],
    compiler_params=pltpu.CompilerParams(
        use_tc_tiling_on_sc=True, needs_layout_passes=False
    ),
)
def _emb_lookup(x_hbm, tok_hbm, pos_hbm, out_hbm, ids_v, idx_v, pair_v,
                pos_v, sem):
    cid = lax.axis_index("c")
    sid = lax.axis_index("s")
    wid = sid * 2 + cid
    base = wid * _RPW
    pos_base = lax.rem(base, _S)

    nrow = _RPW // _G
    x0 = pl.multiple_of(wid * nrow, 8)
    pltpu.sync_copy(x_hbm.at[pl.ds(x0, nrow)], ids_v)

    def _shift(i, carry):
        for j in range(_G // _L):
            sl = pl.ds(j * _L, _L)
            idx_v[i, sl] = lax.shift_right_logical(ids_v[i, sl], 1)
        return carry

    lax.fori_loop(0, nrow, _shift, 0)

    lanes = lax.iota(jnp.int32, _L)

    for k in range(_NCHUNK):
        cps = [
            pltpu.async_copy(
                tok_hbm.at[idx_v.at[k * _NG + g]],
                pair_v.at[pl.ds(g * _G, _G)],
                sem,
            )
            for g in range(_NG)
        ]
        p0 = pl.multiple_of((pos_base + k * _CHUNK) // 2, 8)
        pltpu.sync_copy(pos_hbm.at[pl.ds(p0, _CHUNK // 2)], pos_v)
        for cp in cps:
            cp.wait()

        def _group(g, carry):
            t0 = g * _L
            r = k * (_CHUNK // _G) * _G + t0
            col = pl.multiple_of(lax.rem(r, _G), _L)
            ids16 = ids_v[r // _G, pl.ds(col, _L)]
            t_vec = t0 + lanes
            src_col0 = (ids16 & 1) * _D
            dst_row = lax.shift_right_logical(t_vec, 1)
            dst_col0 = (t_vec & 1) * _D
            for f in range(_D):
                v = plsc.load_gather(pair_v, [t_vec, src_col0 + f])
                p = plsc.load_gather(pos_v, [dst_row, dst_col0 + f])
                plsc.store_scatter(pos_v, [dst_row, dst_col0 + f], p + v)
            return carry

        lax.fori_loop(0, _CHUNK // _L, _group, 0)

        out0 = pl.multiple_of((base + k * _CHUNK) // 2, 8)
        pltpu.sync_copy(pos_v, out_hbm.at[pl.ds(out0, _CHUNK // 2)])


def kernel(x, token_emb, pos_emb):
    idx = x.astype(jnp.int32).reshape(_N // _G, _G)
    tok2 = token_emb.reshape(token_emb.shape[0] // 2, 128)
    pos2 = pos_emb.reshape(_S // 2, 128)
    out = _emb_lookup(idx, tok2, pos2)
    return out.reshape(_B, _S, _D)


# speedup vs baseline: 1.6123x; 1.6123x over previous
import functools

import jax
import jax.numpy as jnp
from jax import lax
from jax.experimental import pallas as pl
from jax.experimental.pallas import tpu as pltpu
from jax.experimental.pallas import tpu_sc as plsc

_B, _S, _D = 16, 2048, 64
_N = _B * _S
_NW = 32
_RPW = _N // _NW
_CHUNK = 512
_NCHUNK = _RPW // _CHUNK
_G = 128
_NG = _CHUNK // _G
_L = 16

_mesh = plsc.VectorSubcoreMesh(core_axis_name="c", subcore_axis_name="s")


@functools.partial(
    pl.kernel,
    mesh=_mesh,
    out_type=jax.ShapeDtypeStruct((_N // 2, 128), jnp.float32),
    scratch_types=[
        pltpu.VMEM((_RPW // _G, _G), jnp.int32),
        pltpu.VMEM((_CHUNK, 128), jnp.float32),
        pltpu.VMEM((_CHUNK // 2, 128), jnp.float32),
        pltpu.SemaphoreType.DMA,
    ],
    compiler_params=pltpu.CompilerParams(
        use_tc_tiling_on_sc=True, needs_layout_passes=False
    ),
)
def _emb_lookup(x_hbm, tok_hbm, pos_hbm, out_hbm, ids_v, gat_v, pos_v, sem):
    cid = lax.axis_index("c")
    sid = lax.axis_index("s")
    wid = sid * 2 + cid
    base = wid * _RPW
    pos_base = lax.rem(base, _S)

    nrow = _RPW // _G
    x0 = pl.multiple_of(wid * nrow, 8)
    pltpu.sync_copy(x_hbm.at[pl.ds(x0, nrow)], ids_v)

    for k in range(_NCHUNK):
        cps = [
            pltpu.async_copy(
                tok_hbm.at[ids_v.at[k * _NG + g]],
                gat_v.at[pl.ds(g * _G, _G)],
                sem,
            )
            for g in range(_NG)
        ]
        p0 = pl.multiple_of((pos_base + k * _CHUNK) // 2, 8)
        pltpu.sync_copy(pos_hbm.at[pl.ds(p0, _CHUNK // 2)], pos_v)
        for cp in cps:
            cp.wait()

        def _add_row(r2, carry):
            for h in range(2):
                for c in range(_D // _L):
                    dst = pl.ds(h * _D + c * _L, _L)
                    src = pl.ds(c * _L, _L)
                    pos_v[r2, dst] = pos_v[r2, dst] + gat_v[2 * r2 + h, src]
            return carry

        lax.fori_loop(0, _CHUNK // 2, _add_row, 0)

        out0 = pl.multiple_of((base + k * _CHUNK) // 2, 8)
        pltpu.sync_copy(pos_v, out_hbm.at[pl.ds(out0, _CHUNK // 2)])


def kernel(x, token_emb, pos_emb):
    idx = x.astype(jnp.int32).reshape(_N // _G, _G)
    tok128 = jnp.concatenate([token_emb, token_emb], axis=1)
    pos2 = pos_emb.reshape(_S // 2, 128)
    out = _emb_lookup(idx, tok128, pos2)
    return out.reshape(_B, _S, _D)
